# baseline (speedup 1.0000x reference)
import jax
import jax.numpy as jnp
from jax import lax
from jax.experimental import pallas as pl
from jax.experimental.pallas import tpu as pltpu
from jax.experimental.pallas import tpu_sc as plsc

EPS = 1e-05

B, C, H, W = 4, 96, 224, 224
HW = H * W
BLK = 3584
NBLK = HW // BLK

NSUB = 16
CCH = 32
NGRP = 4
PPT = HW // NSUB
CHUNK = 128
NCHUNKS = HW // CHUNK
KMAX = (NCHUNKS + NSUB - 1) // NSUB
ZROWS = 196


def _proj_body(x_ref, w_ref, b_ref, xw_ref, dest_ref, off_ref, dst_ref):
    j = pl.program_id(1)
    xblk = x_ref[0]
    oa = jnp.dot(w_ref[...], xblk, preferred_element_type=jnp.float32)
    oa = oa + b_ref[...]
    off_y = oa[0:1] * float(H)
    off_x = oa[1:2] * float(W)
    att = jnp.exp(oa[2:3])

    p = j * BLK + lax.broadcasted_iota(jnp.int32, (1, BLK), 1)
    gy = (p // W).astype(jnp.float32)
    gx = (p - (p // W) * W).astype(jnp.float32)
    dy = jnp.round(gy + off_y).astype(jnp.int32)
    dx = jnp.round(gx + off_x).astype(jnp.int32)
    cy = jnp.clip(dy, 0, H - 1)
    cx = jnp.clip(dx, 0, W - 1)
    dest_ref[0, 0] = (cy * W + cx)[0]

    off_ref[0, 0] = off_y[0]
    off_ref[0, 1] = off_x[0]
    dst_ref[0, 0] = dy[0]
    dst_ref[0, 1] = dx[0]

    q = jnp.concatenate(
        [xblk * att, jnp.broadcast_to(att, (CCH, BLK))], axis=0)
    xw_ref[0] = q.T


def _project(x_flat, Wc8, bc8):
    return pl.pallas_call(
        _proj_body,
        grid=(B, NBLK),
        in_specs=[
            pl.BlockSpec((1, C, BLK), lambda b, j: (b, 0, j)),
            pl.BlockSpec((8, C), lambda b, j: (0, 0)),
            pl.BlockSpec((8, 1), lambda b, j: (0, 0)),
        ],
        out_specs=[
            pl.BlockSpec((1, BLK, 128), lambda b, j: (b, j, 0)),
            pl.BlockSpec((1, 1, BLK), lambda b, j: (b, 0, j)),
            pl.BlockSpec((1, 2, BLK), lambda b, j: (b, 0, j)),
            pl.BlockSpec((1, 2, BLK), lambda b, j: (b, 0, j)),
        ],
        out_shape=[
            jax.ShapeDtypeStruct((B, HW, 128), jnp.float32),
            jax.ShapeDtypeStruct((B, 1, HW), jnp.int32),
            jax.ShapeDtypeStruct((B, 2, HW), jnp.float32),
            jax.ShapeDtypeStruct((B, 2, HW), jnp.int32),
        ],
    )(x_flat, Wc8, bc8)


def _scatter_body(xw4, dest, z2, feat_out, acc, zv2, idxb, rowb, dumpb):
    core = lax.axis_index("c")
    sid = lax.axis_index("s")

    pltpu.sync_copy(z2, zv2)

    def run_task(b, ch):
        for m in range(PPT // ZROWS):
            pltpu.sync_copy(
                zv2, acc.at[pl.ds(sid * PPT + m * ZROWS, ZROWS)])
        plsc.subcore_barrier()

        for k in range(KMAX):
            g = sid + k * NSUB

            @pl.when(g < NCHUNKS)
            def _():
                pltpu.sync_copy(dest.at[pl.ds(b * HW + g * CHUNK, CHUNK)],
                                idxb.at[0])
                pltpu.sync_copy(xw4.at[b, pl.ds(g * CHUNK, CHUNK), ch], rowb)
                pltpu.sync_copy(rowb, acc.at[idxb.at[0]], add=True)

        plsc.subcore_barrier()

        for m in range(PPT // ZROWS):
            off = sid * PPT + m * ZROWS
            pltpu.sync_copy(acc.at[pl.ds(off, ZROWS)], dumpb)
            pltpu.sync_copy(dumpb, feat_out.at[b, pl.ds(off, ZROWS), ch])
        plsc.subcore_barrier()

    for t in range(8):
        run_task(core * 2 + t // 4, t % 4)


def _scatter(xw4g, dest):
    z2 = jnp.zeros((ZROWS, CCH), jnp.float32)
    mesh = plsc.VectorSubcoreMesh(core_axis_name="c", subcore_axis_name="s")
    kern = pl.kernel(
        _scatter_body,
        mesh=mesh,
        out_type=[
            jax.ShapeDtypeStruct((B, HW, NGRP, CCH), jnp.float32),
        ],
        scratch_types=[
            pltpu.VMEM_SHARED((HW, CCH), jnp.float32),
            pltpu.VMEM((ZROWS, CCH), jnp.float32),
            pltpu.VMEM((1, CHUNK), jnp.int32),
            pltpu.VMEM((CHUNK, CCH), jnp.float32),
            pltpu.VMEM((ZROWS, CCH), jnp.float32),
        ],
        compiler_params=pltpu.CompilerParams(use_tc_tiling_on_sc=False),
    )
    return kern(xw4g, dest, z2)


def _final_body(feat_ref, out_ref):
    y = feat_ref[0].T
    r = 1.0 / (y[C:C + 1] + EPS)
    out_ref[0] = y[0:C] * r


def _finalize(featacc):
    return pl.pallas_call(
        _final_body,
        grid=(B, NBLK),
        in_specs=[
            pl.BlockSpec((1, BLK, 128), lambda b, j: (b, j, 0)),
        ],
        out_specs=pl.BlockSpec((1, C, BLK), lambda b, j: (b, 0, j)),
        out_shape=jax.ShapeDtypeStruct((B, C, HW), jnp.float32),
    )(featacc)


def kernel(x, Wc, bc):
    x_flat = x.reshape(B, C, HW)
    Wc8 = jnp.zeros((8, C), jnp.float32).at[:3].set(Wc)
    bc8 = jnp.zeros((8, 1), jnp.float32).at[:3, 0].set(bc)

    xw4, dest3, offset, destination = _project(x_flat, Wc8, bc8)
    dest = dest3.reshape(B * HW)
    (featacc,) = _scatter(xw4.reshape(B, HW, NGRP, CCH), dest)
    out = _finalize(featacc.reshape(B, HW, 128))

    return (out.reshape(B, C, H, W),
            offset.reshape(B, 2, H, W),
            destination.reshape(B, 2, H, W))


# speedup vs baseline: 2.3057x; 2.3057x over previous
import jax
import jax.numpy as jnp
from jax import lax
from jax.experimental import pallas as pl
from jax.experimental.pallas import tpu as pltpu
from jax.experimental.pallas import tpu_sc as plsc

EPS = 1e-05

B, C, H, W = 4, 96, 224, 224
HW = H * W
BLK = 3584
NBLK = HW // BLK

NSUB = 16
CCH = 32
NGRP = 4
PPT = HW // NSUB
CHUNK = 128
NCHUNKS = HW // CHUNK
KMAX = (NCHUNKS + NSUB - 1) // NSUB
ZROWS = 196


def _proj_body(x_ref, w_ref, b_ref, xw_ref, dest_ref, off_ref, dst_ref):
    j = pl.program_id(1)
    xblk = x_ref[0]
    oa = jnp.dot(w_ref[...], xblk, preferred_element_type=jnp.float32)
    oa = oa + b_ref[...]
    off_y = oa[0:1] * float(H)
    off_x = oa[1:2] * float(W)
    att = jnp.exp(oa[2:3])

    p = j * BLK + lax.broadcasted_iota(jnp.int32, (1, BLK), 1)
    gy = (p // W).astype(jnp.float32)
    gx = (p - (p // W) * W).astype(jnp.float32)
    dy = jnp.round(gy + off_y).astype(jnp.int32)
    dx = jnp.round(gx + off_x).astype(jnp.int32)
    cy = jnp.clip(dy, 0, H - 1)
    cx = jnp.clip(dx, 0, W - 1)
    dest_ref[0, 0] = (cy * W + cx)[0]

    off_ref[0, 0] = off_y[0]
    off_ref[0, 1] = off_x[0]
    dst_ref[0, 0] = dy[0]
    dst_ref[0, 1] = dx[0]

    q = jnp.concatenate(
        [xblk * att, jnp.broadcast_to(att, (CCH, BLK))], axis=0)
    xw_ref[0] = q.T


def _project(x_flat, Wc8, bc8):
    return pl.pallas_call(
        _proj_body,
        grid=(B, NBLK),
        in_specs=[
            pl.BlockSpec((1, C, BLK), lambda b, j: (b, 0, j)),
            pl.BlockSpec((8, C), lambda b, j: (0, 0)),
            pl.BlockSpec((8, 1), lambda b, j: (0, 0)),
        ],
        out_specs=[
            pl.BlockSpec((1, BLK, 128), lambda b, j: (b, j, 0)),
            pl.BlockSpec((1, 1, BLK), lambda b, j: (b, 0, j)),
            pl.BlockSpec((1, 2, BLK), lambda b, j: (b, 0, j)),
            pl.BlockSpec((1, 2, BLK), lambda b, j: (b, 0, j)),
        ],
        out_shape=[
            jax.ShapeDtypeStruct((B, HW, 128), jnp.float32),
            jax.ShapeDtypeStruct((B, 1, HW), jnp.int32),
            jax.ShapeDtypeStruct((B, 2, HW), jnp.float32),
            jax.ShapeDtypeStruct((B, 2, HW), jnp.int32),
        ],
    )(x_flat, Wc8, bc8)


def _scatter_body(xw4, dest, z2, feat_out, acc, zv2, idxb, rowb, dumpb):
    core = lax.axis_index("c")
    sid = lax.axis_index("s")

    pltpu.sync_copy(z2, zv2)

    def run_task(b, ch):
        for m in range(PPT // ZROWS):
            pltpu.sync_copy(
                zv2, acc.at[pl.ds(sid * PPT + m * ZROWS, ZROWS)])
        plsc.subcore_barrier()

        for k in range(KMAX):
            g = sid + k * NSUB

            @pl.when(g < NCHUNKS)
            def _():
                pltpu.sync_copy(dest.at[pl.ds(b * HW + g * CHUNK, CHUNK)],
                                idxb.at[0])
                pltpu.sync_copy(
                    xw4.at[b, pl.ds(g * CHUNK, CHUNK),
                           pl.ds(ch * CCH, CCH)], rowb)
                pltpu.sync_copy(rowb, acc.at[idxb.at[0]], add=True)

        plsc.subcore_barrier()

        for m in range(PPT // ZROWS):
            off = sid * PPT + m * ZROWS
            pltpu.sync_copy(acc.at[pl.ds(off, ZROWS)], dumpb)
            pltpu.sync_copy(
                dumpb,
                feat_out.at[b, pl.ds(off, ZROWS), pl.ds(ch * CCH, CCH)])
        plsc.subcore_barrier()

    for t in range(8):
        run_task(core * 2 + t // 4, t % 4)


def _scatter(xw4g, dest):
    z2 = jnp.zeros((ZROWS, CCH), jnp.float32)
    mesh = plsc.VectorSubcoreMesh(core_axis_name="c", subcore_axis_name="s")
    kern = pl.kernel(
        _scatter_body,
        mesh=mesh,
        out_type=[
            jax.ShapeDtypeStruct((B, HW, 128), jnp.float32),
        ],
        scratch_types=[
            pltpu.VMEM_SHARED((HW, CCH), jnp.float32),
            pltpu.VMEM((ZROWS, CCH), jnp.float32),
            pltpu.VMEM((1, CHUNK), jnp.int32),
            pltpu.VMEM((CHUNK, CCH), jnp.float32),
            pltpu.VMEM((ZROWS, CCH), jnp.float32),
        ],
        compiler_params=pltpu.CompilerParams(use_tc_tiling_on_sc=False),
    )
    return kern(xw4g, dest, z2)


def _final_body(feat_ref, out_ref):
    y = feat_ref[0].T
    r = 1.0 / (y[C:C + 1] + EPS)
    out_ref[0] = y[0:C] * r


def _finalize(featacc):
    return pl.pallas_call(
        _final_body,
        grid=(B, NBLK),
        in_specs=[
            pl.BlockSpec((1, BLK, 128), lambda b, j: (b, j, 0)),
        ],
        out_specs=pl.BlockSpec((1, C, BLK), lambda b, j: (b, 0, j)),
        out_shape=jax.ShapeDtypeStruct((B, C, HW), jnp.float32),
    )(featacc)


def kernel(x, Wc, bc):
    x_flat = x.reshape(B, C, HW)
    Wc8 = jnp.zeros((8, C), jnp.float32).at[:3].set(Wc)
    bc8 = jnp.zeros((8, 1), jnp.float32).at[:3, 0].set(bc)

    xw4, dest3, offset, destination = _project(x_flat, Wc8, bc8)
    dest = dest3.reshape(B * HW)
    (featacc,) = _scatter(xw4, dest)
    out = _finalize(featacc)

    return (out.reshape(B, C, H, W),
            offset.reshape(B, 2, H, W),
            destination.reshape(B, 2, H, W))


# speedup vs baseline: 2.3350x; 1.0127x over previous
import jax
import jax.numpy as jnp
from jax import lax
from jax.experimental import pallas as pl
from jax.experimental.pallas import tpu as pltpu
from jax.experimental.pallas import tpu_sc as plsc

EPS = 1e-05

B, C, H, W = 4, 96, 224, 224
HW = H * W
BLK = 3584
NBLK = HW // BLK

NSUB = 16
CCH = 32
NGRP = 4
PPT = HW // NSUB
CHUNK = 128
NCHUNKS = HW // CHUNK
KMAX = (NCHUNKS + NSUB - 1) // NSUB
ZROWS = 196


ROWS = 8
RBLK = ROWS * W


def _proj_body(x_ref, w_ref, b_ref, xw_ref, dest_ref, off_ref, dst_ref):
    j = pl.program_id(1)
    xblk = x_ref[0]
    for r in range(ROWS):
        xr = xblk[:, r, :]
        oa = jnp.dot(w_ref[...], xr, preferred_element_type=jnp.float32)
        oa = oa + b_ref[...]
        off_y = oa[0:1] * float(H)
        off_x = oa[1:2] * float(W)
        att = jnp.exp(oa[2:3])

        gy = (j * ROWS + r).astype(jnp.float32)
        gx = lax.broadcasted_iota(jnp.int32, (1, W), 1).astype(jnp.float32)
        dy = jnp.round(gy + off_y).astype(jnp.int32)
        dx = jnp.round(gx + off_x).astype(jnp.int32)
        cy = jnp.clip(dy, 0, H - 1)
        cx = jnp.clip(dx, 0, W - 1)
        dest_ref[0, 0, r] = (cy * W + cx)[0]

        off_ref[0, 0, r] = off_y[0]
        off_ref[0, 1, r] = off_x[0]
        dst_ref[0, 0, r] = dy[0]
        dst_ref[0, 1, r] = dx[0]

        q = jnp.concatenate(
            [xr * att, jnp.broadcast_to(att, (CCH, W))], axis=0)
        xw_ref[0, pl.ds(r * W, W)] = q.T


def _project(x, Wc8, bc8):
    return pl.pallas_call(
        _proj_body,
        grid=(B, H // ROWS),
        in_specs=[
            pl.BlockSpec((1, C, ROWS, W), lambda b, j: (b, 0, j, 0)),
            pl.BlockSpec((8, C), lambda b, j: (0, 0)),
            pl.BlockSpec((8, 1), lambda b, j: (0, 0)),
        ],
        out_specs=[
            pl.BlockSpec((1, RBLK, 128), lambda b, j: (b, j, 0)),
            pl.BlockSpec((1, 1, ROWS, W), lambda b, j: (b, 0, j, 0)),
            pl.BlockSpec((1, 2, ROWS, W), lambda b, j: (b, 0, j, 0)),
            pl.BlockSpec((1, 2, ROWS, W), lambda b, j: (b, 0, j, 0)),
        ],
        out_shape=[
            jax.ShapeDtypeStruct((B, HW, 128), jnp.float32),
            jax.ShapeDtypeStruct((B, 1, H, W), jnp.int32),
            jax.ShapeDtypeStruct((B, 2, H, W), jnp.float32),
            jax.ShapeDtypeStruct((B, 2, H, W), jnp.int32),
        ],
    )(x, Wc8, bc8)


def _scatter_body(xw4, dest, z2, feat_out, acc, zv2, idxb, rowb, dumpb):
    core = lax.axis_index("c")
    sid = lax.axis_index("s")

    pltpu.sync_copy(z2, zv2)

    def run_task(b, ch):
        for m in range(PPT // ZROWS):
            pltpu.sync_copy(
                zv2, acc.at[pl.ds(sid * PPT + m * ZROWS, ZROWS)])
        plsc.subcore_barrier()

        for k in range(KMAX):
            g = sid + k * NSUB

            @pl.when(g < NCHUNKS)
            def _():
                pltpu.sync_copy(dest.at[pl.ds(b * HW + g * CHUNK, CHUNK)],
                                idxb.at[0])
                pltpu.sync_copy(
                    xw4.at[b, pl.ds(g * CHUNK, CHUNK),
                           pl.ds(ch * CCH, CCH)], rowb)
                pltpu.sync_copy(rowb, acc.at[idxb.at[0]], add=True)

        plsc.subcore_barrier()

        for m in range(PPT // ZROWS):
            off = sid * PPT + m * ZROWS
            pltpu.sync_copy(acc.at[pl.ds(off, ZROWS)], dumpb)
            pltpu.sync_copy(
                dumpb,
                feat_out.at[b, pl.ds(off, ZROWS), pl.ds(ch * CCH, CCH)])
        plsc.subcore_barrier()

    for t in range(8):
        run_task(core * 2 + t // 4, t % 4)


def _scatter(xw4g, dest):
    z2 = jnp.zeros((ZROWS, CCH), jnp.float32)
    mesh = plsc.VectorSubcoreMesh(core_axis_name="c", subcore_axis_name="s")
    kern = pl.kernel(
        _scatter_body,
        mesh=mesh,
        out_type=[
            jax.ShapeDtypeStruct((B, HW, 128), jnp.float32),
        ],
        scratch_types=[
            pltpu.VMEM_SHARED((HW, CCH), jnp.float32),
            pltpu.VMEM((ZROWS, CCH), jnp.float32),
            pltpu.VMEM((1, CHUNK), jnp.int32),
            pltpu.VMEM((CHUNK, CCH), jnp.float32),
            pltpu.VMEM((ZROWS, CCH), jnp.float32),
        ],
        compiler_params=pltpu.CompilerParams(use_tc_tiling_on_sc=False),
    )
    return kern(xw4g, dest, z2)


def _final_body(feat_ref, out_ref):
    y = feat_ref[0].T
    r = 1.0 / (y[C:C + 1] + EPS)
    out_ref[0] = y[0:C] * r


def _finalize(featacc):
    return pl.pallas_call(
        _final_body,
        grid=(B, NBLK),
        in_specs=[
            pl.BlockSpec((1, BLK, 128), lambda b, j: (b, j, 0)),
        ],
        out_specs=pl.BlockSpec((1, C, BLK), lambda b, j: (b, 0, j)),
        out_shape=jax.ShapeDtypeStruct((B, C, HW), jnp.float32),
    )(featacc)


def kernel(x, Wc, bc):
    Wc8 = jnp.zeros((8, C), jnp.float32).at[:3].set(Wc)
    bc8 = jnp.zeros((8, 1), jnp.float32).at[:3, 0].set(bc)

    xw4, dest3, offset, destination = _project(x, Wc8, bc8)
    dest = dest3.reshape(B * HW)
    (featacc,) = _scatter(xw4, dest)
    out = _finalize(featacc)

    return (out.reshape(B, C, H, W), offset, destination)


# speedup vs baseline: 2.4742x; 1.0596x over previous
import jax
import jax.numpy as jnp
from jax import lax
from jax.experimental import pallas as pl
from jax.experimental.pallas import tpu as pltpu
from jax.experimental.pallas import tpu_sc as plsc

EPS = 1e-05

B, C, H, W = 4, 96, 224, 224
HW = H * W
BLK = 3584
NBLK = HW // BLK

NSUB = 16
CCH = 32
NGRP = 4
PPT = HW // NSUB
CHUNK = 128
NCHUNKS = HW // CHUNK
KMAX = (NCHUNKS + NSUB - 1) // NSUB
ZROWS = 196


ROWS = 8
RBLK = ROWS * W


def _proj_body(x_ref, w_ref, b_ref, xw_ref, dest_ref, off_ref, dst_ref):
    j = pl.program_id(1)
    xblk = x_ref[0]
    for r in range(ROWS):
        xr = xblk[:, r, :]
        oa = jnp.dot(w_ref[...], xr, preferred_element_type=jnp.float32)
        oa = oa + b_ref[...]
        off_y = oa[0:1] * float(H)
        off_x = oa[1:2] * float(W)
        att = jnp.exp(oa[2:3])

        gy = (j * ROWS + r).astype(jnp.float32)
        gx = lax.broadcasted_iota(jnp.int32, (1, W), 1).astype(jnp.float32)
        dy = jnp.round(gy + off_y).astype(jnp.int32)
        dx = jnp.round(gx + off_x).astype(jnp.int32)
        cy = jnp.clip(dy, 0, H - 1)
        cx = jnp.clip(dx, 0, W - 1)
        dest_ref[0, 0, r] = (cy * W + cx)[0]

        off_ref[0, 0, r] = off_y[0]
        off_ref[0, 1, r] = off_x[0]
        dst_ref[0, 0, r] = dy[0]
        dst_ref[0, 1, r] = dx[0]

        q = jnp.concatenate(
            [xr * att, jnp.broadcast_to(att, (CCH, W))], axis=0)
        xw_ref[0, pl.ds(r * W, W)] = q.T


def _project(x, Wc8, bc8):
    return pl.pallas_call(
        _proj_body,
        grid=(B, H // ROWS),
        in_specs=[
            pl.BlockSpec((1, C, ROWS, W), lambda b, j: (b, 0, j, 0)),
            pl.BlockSpec((8, C), lambda b, j: (0, 0)),
            pl.BlockSpec((8, 1), lambda b, j: (0, 0)),
        ],
        out_specs=[
            pl.BlockSpec((1, RBLK, 128), lambda b, j: (b, j, 0)),
            pl.BlockSpec((1, 1, ROWS, W), lambda b, j: (b, 0, j, 0)),
            pl.BlockSpec((1, 2, ROWS, W), lambda b, j: (b, 0, j, 0)),
            pl.BlockSpec((1, 2, ROWS, W), lambda b, j: (b, 0, j, 0)),
        ],
        out_shape=[
            jax.ShapeDtypeStruct((B, HW, 128), jnp.float32),
            jax.ShapeDtypeStruct((B, 1, H, W), jnp.int32),
            jax.ShapeDtypeStruct((B, 2, H, W), jnp.float32),
            jax.ShapeDtypeStruct((B, 2, H, W), jnp.int32),
        ],
    )(x, Wc8, bc8)


def _scatter_body(xw4, dest, z2, feat_out, acc, zv2, idxb, rowb, dumpb):
    core = lax.axis_index("c")
    sid = lax.axis_index("s")

    pltpu.sync_copy(z2, zv2)

    def run_task(b, ch):
        for m in range(PPT // ZROWS):
            pltpu.sync_copy(
                zv2, acc.at[pl.ds(sid * PPT + m * ZROWS, ZROWS)])
        plsc.subcore_barrier()

        for k in range(KMAX):
            g = sid + k * NSUB

            @pl.when(g < NCHUNKS)
            def _():
                pltpu.sync_copy(dest.at[pl.ds(b * HW + g * CHUNK, CHUNK)],
                                idxb.at[0])
                pltpu.sync_copy(
                    xw4.at[b, pl.ds(g * CHUNK, CHUNK),
                           pl.ds(ch * CCH, CCH)], rowb)
                pltpu.sync_copy(rowb, acc.at[idxb.at[0]], add=True)

        plsc.subcore_barrier()

        for m in range(PPT // ZROWS):
            off = sid * PPT + m * ZROWS
            pltpu.sync_copy(acc.at[pl.ds(off, ZROWS)], dumpb)
            pltpu.sync_copy(
                dumpb,
                feat_out.at[b, pl.ds(off, ZROWS), pl.ds(ch * CCH, CCH)])
        plsc.subcore_barrier()

    for t in range(8):
        run_task(core * 2 + t // 4, t % 4)


def _scatter(xw4g, dest):
    z2 = jnp.zeros((ZROWS, CCH), jnp.float32)
    mesh = plsc.VectorSubcoreMesh(core_axis_name="c", subcore_axis_name="s")
    kern = pl.kernel(
        _scatter_body,
        mesh=mesh,
        out_type=[
            jax.ShapeDtypeStruct((B, HW, 128), jnp.float32),
        ],
        scratch_types=[
            pltpu.VMEM_SHARED((HW, CCH), jnp.float32),
            pltpu.VMEM((ZROWS, CCH), jnp.float32),
            pltpu.VMEM((1, CHUNK), jnp.int32),
            pltpu.VMEM((CHUNK, CCH), jnp.float32),
            pltpu.VMEM((ZROWS, CCH), jnp.float32),
        ],
        compiler_params=pltpu.CompilerParams(use_tc_tiling_on_sc=False),
    )
    return kern(xw4g, dest, z2)


def _final_body(feat_ref, out_ref):
    for r in range(ROWS):
        y = feat_ref[0, pl.ds(r * W, W)].T
        rr = 1.0 / (y[C:C + 1] + EPS)
        out_ref[0, :, r, :] = y[0:C] * rr


def _finalize(featacc):
    return pl.pallas_call(
        _final_body,
        grid=(B, H // ROWS),
        in_specs=[
            pl.BlockSpec((1, RBLK, 128), lambda b, j: (b, j, 0)),
        ],
        out_specs=pl.BlockSpec((1, C, ROWS, W), lambda b, j: (b, 0, j, 0)),
        out_shape=jax.ShapeDtypeStruct((B, C, H, W), jnp.float32),
    )(featacc)


def kernel(x, Wc, bc):
    Wc8 = jnp.zeros((8, C), jnp.float32).at[:3].set(Wc)
    bc8 = jnp.zeros((8, 1), jnp.float32).at[:3, 0].set(bc)

    xw4, dest3, offset, destination = _project(x, Wc8, bc8)
    dest = dest3.reshape(B * HW)
    (featacc,) = _scatter(xw4, dest)
    out = _finalize(featacc)

    return (out, offset, destination)
